# baseline (speedup 1.0000x reference)
import functools

import jax
import jax.numpy as jnp
from jax import lax
from jax.experimental import pallas as pl
from jax.experimental.pallas import tpu as pltpu
from jax.experimental.pallas import tpu_sc as plsc

N = 10000
D = 128
E = 320000
H = 20
OH = 20
XDIM = 20

W = 32
NC = 2
NS = 16
NW = NC * NS
SEG_CH = 512
SEG_CPW = 20
DOT_CH = 256
DOT_CPW = 40
EPW = SEG_CPW * SEG_CH
E_PAD = NW * EPW
N_ACC = N + 112
ZR = N_ACC // NS

BN = 1000
f32 = jnp.float32

def _sc_mesh():
    return plsc.VectorSubcoreMesh(core_axis_name="c", subcore_axis_name="s",
                                  num_cores=NC, num_subcores=NS)


def _segsum_body(p_hbm, src_hbm, dst_hbm, zero_hbm, out_hbm,
                 src_v, dst_v, rows_v, acc_sh, *sems):
    gs = sems[:4]
    ss = sems[4:]
    c = lax.axis_index("c")
    s = lax.axis_index("s")
    w = c * NS + s
    pltpu.sync_copy(zero_hbm, acc_sh.at[pl.ds(s * ZR, ZR)])
    pltpu.sync_copy(src_hbm.at[w], src_v)
    pltpu.sync_copy(dst_hbm.at[w], dst_v)
    plsc.subcore_barrier()

    def gather(j, b):
        pltpu.async_copy(p_hbm.at[src_v.at[j]], rows_v.at[b], gs[b])

    def gather_wait(b):
        pltpu.make_async_copy(p_hbm.at[src_v.at[0]], rows_v.at[b], gs[b]).wait()

    def scatter(j, b):
        pltpu.async_copy(rows_v.at[b], acc_sh.at[dst_v.at[j]], ss[b], add=True)

    def scatter_wait(b):
        pltpu.make_async_copy(rows_v.at[b], acc_sh.at[dst_v.at[0]], ss[b]).wait()

    gather(0, 0)

    def pair(jj, carry):
        for b in range(2):
            j = jj * 2 + b
            gather_wait(b)
            gather(jnp.minimum(j + 1, SEG_CPW - 1), 1 - b)
            pltpu.sync_copy(rows_v.at[b], acc_sh.at[dst_v.at[j]], add=True)
        return carry

    lax.fori_loop(0, SEG_CPW // 2, pair, 0)
    gather_wait(0)
    plsc.subcore_barrier()
    pltpu.sync_copy(acc_sh.at[pl.ds(s * ZR, ZR)],
                    out_hbm.at[c].at[pl.ds(s * ZR, ZR)])


@functools.cache
def _segsum_kernel():
    return pl.kernel(
        _segsum_body,
        out_type=jax.ShapeDtypeStruct((NC, N_ACC, W), f32),
        mesh=_sc_mesh(),
        scratch_types=[
            pltpu.VMEM((SEG_CPW, SEG_CH), jnp.int32),
            pltpu.VMEM((SEG_CPW, SEG_CH), jnp.int32),
            pltpu.VMEM((4, SEG_CH, W), f32),
            pltpu.VMEM_SHARED((N_ACC, W), f32),
        ] + [pltpu.SemaphoreType.DMA] * 8,
        compiler_params=pltpu.CompilerParams(use_tc_tiling_on_sc=False,
                                             needs_layout_passes=False),
    )


def _segsum(p, src, dst, zero):
    return _segsum_kernel()(p, src, dst, zero)


def _edgedot_body(zz_hbm, src_hbm, dst_hbm, out_hbm,
                  src_v, dst_v, ab_v, sim_v, *sems):
    ga = sems[0:2]
    gb = sems[2:4]
    os_ = sems[4:6]
    c = lax.axis_index("c")
    s = lax.axis_index("s")
    w = c * NS + s
    pltpu.sync_copy(src_hbm.at[w], src_v)
    pltpu.sync_copy(dst_hbm.at[w], dst_v)
    lane = lax.iota(jnp.int32, 16)

    def gathers(j, b):
        pltpu.async_copy(zz_hbm.at[src_v.at[j]], ab_v.at[b].at[0], ga[b])
        pltpu.async_copy(zz_hbm.at[dst_v.at[j]], ab_v.at[b].at[1], gb[b])

    def gathers_wait(b):
        pltpu.make_async_copy(zz_hbm.at[src_v.at[0]], ab_v.at[b].at[0], ga[b]).wait()
        pltpu.make_async_copy(zz_hbm.at[dst_v.at[0]], ab_v.at[b].at[1], gb[b]).wait()

    def out_wait(b):
        pltpu.make_async_copy(sim_v.at[b], out_hbm.at[pl.ds(0, DOT_CH)], os_[b]).wait()

    gathers(0, 0)

    def pair(jj, carry):
        for b in range(2):
            j = jj * 2 + b
            gathers_wait(b)
            gathers(jnp.minimum(j + 1, DOT_CPW - 1), 1 - b)
            a_rows = ab_v.at[b].at[0]
            b_rows = ab_v.at[b].at[1]
            for g in range(DOT_CH // 16):
                rows = lane + (g * 16)
                acc = jnp.zeros((16,), f32)
                for f in range(OH):
                    col = jnp.full((16,), f, jnp.int32)
                    acc = acc + (plsc.load_gather(a_rows, (rows, col))
                                 * plsc.load_gather(b_rows, (rows, col)))
                sim_v[b, pl.ds(g * 16, 16)] = 1.0 / (1.0 + jnp.exp(-acc))
            pltpu.sync_copy(sim_v.at[b],
                            out_hbm.at[pl.ds((j * NW + w) * DOT_CH, DOT_CH)])
        return carry

    lax.fori_loop(0, DOT_CPW // 2, pair, 0)
    gathers_wait(0)


@functools.cache
def _edgedot_kernel():
    return pl.kernel(
        _edgedot_body,
        out_type=jax.ShapeDtypeStruct((E_PAD,), f32),
        mesh=_sc_mesh(),
        scratch_types=[
            pltpu.VMEM((DOT_CPW, DOT_CH), jnp.int32),
            pltpu.VMEM((DOT_CPW, DOT_CH), jnp.int32),
            pltpu.VMEM((2, 2, DOT_CH, W), f32),
            pltpu.VMEM((2, DOT_CH), f32),
        ] + [pltpu.SemaphoreType.DMA] * 6,
        compiler_params=pltpu.CompilerParams(use_tc_tiling_on_sc=False,
                                             needs_layout_passes=False),
    )


def _edgedot(zz, src, dst):
    return _edgedot_kernel()(zz, src, dst)


def _ones_col():
    col = lax.broadcasted_iota(jnp.int32, (1, W), 1)
    return jnp.where(col == H, 1.0, 0.0).astype(f32)


def _prep0_body(z_ref, wl_ref, wr_ref, b_ref, p_ref, r_ref):
    zb = z_ref[...]
    p_ref[...] = jnp.dot(zb, wl_ref[...], preferred_element_type=f32) + _ones_col()
    r_ref[...] = jnp.dot(zb, wr_ref[...], preferred_element_type=f32) + b_ref[...]


def _combine(pa, pb, r):
    ssum = pa + pb
    col = lax.broadcasted_iota(jnp.int32, (1, W), 1)
    cnt = jnp.sum(jnp.where(col == H, ssum, 0.0), axis=1, keepdims=True)
    mean = ssum / jnp.maximum(cnt, 1.0)
    return jnp.maximum(mean + r, 0.0)


def _comb_prep_body(pa_ref, pb_ref, r_ref, wl_ref, wr_ref, b_ref, p_ref, rn_ref):
    h = _combine(pa_ref[...], pb_ref[...], r_ref[...])
    p_ref[...] = jnp.dot(h, wl_ref[...], preferred_element_type=f32) + _ones_col()
    rn_ref[...] = jnp.dot(h, wr_ref[...], preferred_element_type=f32) + b_ref[...]


def _zz_body(pa_ref, pb_ref, r_ref, lw_ref, lb_ref, zz_ref):
    h = _combine(pa_ref[...], pb_ref[...], r_ref[...])
    zz_ref[...] = jnp.dot(h, lw_ref[...], preferred_element_type=f32) + lb_ref[...]


def _dec_body(z_ref, zz_ref, w0a_ref, w0b_ref, b0_ref, w1_ref, b1_ref,
              w2_ref, b2_ref, w3_ref, b3_ref, w4_ref, b4_ref, out_ref):
    x = jnp.maximum(jnp.dot(z_ref[...], w0a_ref[...], preferred_element_type=f32)
                    + jnp.dot(zz_ref[...], w0b_ref[...], preferred_element_type=f32)
                    + b0_ref[...], 0.0)
    for wr, br in ((w1_ref, b1_ref), (w2_ref, b2_ref), (w3_ref, b3_ref)):
        x = jnp.maximum(jnp.dot(x, wr[...], preferred_element_type=f32) + br[...], 0.0)
    lg = jnp.dot(x, w4_ref[...], preferred_element_type=f32) + b4_ref[...]
    col = lax.broadcasted_iota(jnp.int32, (1, W), 1)
    neg = jnp.where(col < XDIM, lg, -1e30)
    m = jnp.max(neg, axis=1, keepdims=True)
    ex = jnp.where(col < XDIM, jnp.exp(neg - m), 0.0)
    out_ref[...] = (neg - m) - jnp.log(jnp.sum(ex, axis=1, keepdims=True))


def _full(shape):
    return pl.BlockSpec(shape, lambda i: (0, 0))


def _rows(width):
    return pl.BlockSpec((BN, width), lambda i: (i, 0))


_GRID = (N // BN,)


def _call_prep0(z, wl, wr, b):
    return pl.pallas_call(
        _prep0_body, grid=_GRID,
        in_specs=[_rows(D), _full((D, W)), _full((D, W)), _full((1, W))],
        out_specs=[_rows(W), _rows(W)],
        out_shape=[jax.ShapeDtypeStruct((N, W), f32)] * 2,
    )(z, wl, wr, b)


def _call_comb_prep(pa, pb, r, wl, wr, b):
    return pl.pallas_call(
        _comb_prep_body, grid=_GRID,
        in_specs=[_rows(W), _rows(W), _rows(W),
                  _full((W, W)), _full((W, W)), _full((1, W))],
        out_specs=[_rows(W), _rows(W)],
        out_shape=[jax.ShapeDtypeStruct((N, W), f32)] * 2,
    )(pa, pb, r, wl, wr, b)


def _call_zz(pa, pb, r, lw, lb):
    return pl.pallas_call(
        _zz_body, grid=_GRID,
        in_specs=[_rows(W), _rows(W), _rows(W), _full((W, W)), _full((1, W))],
        out_specs=_rows(W),
        out_shape=jax.ShapeDtypeStruct((N, W), f32),
    )(pa, pb, r, lw, lb)


def _call_dec(z, zz, w0a, w0b, b0, w1, b1, w2, b2, w3, b3, w4, b4):
    return pl.pallas_call(
        _dec_body, grid=_GRID,
        in_specs=[_rows(D), _rows(W),
                  _full((D, W)), _full((W, W)), _full((1, W)),
                  _full((W, W)), _full((1, W)),
                  _full((W, W)), _full((1, W)),
                  _full((W, W)), _full((1, W)),
                  _full((W, W)), _full((1, W))],
        out_specs=_rows(W),
        out_shape=jax.ShapeDtypeStruct((N, W), f32),
    )(z, zz, w0a, w0b, b0, w1, b1, w2, b2, w3, b3, w4, b4)


def _pad_w(w, rows, cols):
    return jnp.zeros((rows, cols), f32).at[:w.shape[0], :w.shape[1]].set(w)


def _pad_b(b, cols):
    return jnp.zeros((1, cols), f32).at[0, :b.shape[0]].set(b)


def _prep_edges(src, dst, spread_dump, ch, cpw):
    pad = E_PAD - E
    if spread_dump:
        fill = (N + jnp.arange(pad, dtype=jnp.int32) % (N_ACC - N))
    else:
        fill = jnp.zeros((pad,), jnp.int32)
    srcp = jnp.concatenate([src, jnp.zeros((pad,), jnp.int32)])
    dstp = jnp.concatenate([dst, fill])
    srcp = srcp.reshape(cpw, NW, ch).transpose(1, 0, 2)
    dstp = dstp.reshape(cpw, NW, ch).transpose(1, 0, 2)
    return srcp, dstp


def kernel(z, edge_index, backbones, Wl0, Wr0, b0, Wl1, Wr1, b1, Wl2, Wr2, b2,
           linW, linB, dW0, db0, dW1, db1, dW2, db2, dW3, db3, dW4, db4):
    sb, db = _prep_edges(backbones[0], backbones[1], True, SEG_CH, SEG_CPW)
    se, de = _prep_edges(edge_index[0], edge_index[1], False, DOT_CH, DOT_CPW)
    zero_rows = jnp.zeros((ZR, W), f32)

    p, r = _call_prep0(z, _pad_w(Wl0, D, W), _pad_w(Wr0, D, W), _pad_b(b0, W))
    part = _segsum(p, sb, db, zero_rows)
    p, r = _call_comb_prep(part[0], part[1], r,
                           _pad_w(Wl1, W, W), _pad_w(Wr1, W, W), _pad_b(b1, W))
    part = _segsum(p, sb, db, zero_rows)
    p, r = _call_comb_prep(part[0], part[1], r,
                           _pad_w(Wl2, W, W), _pad_w(Wr2, W, W), _pad_b(b2, W))
    part = _segsum(p, sb, db, zero_rows)
    zz = _call_zz(part[0], part[1], r, _pad_w(linW, W, W), _pad_b(linB, W))

    sim = _edgedot(zz, se, de)
    x_r = _call_dec(z, zz,
                    _pad_w(dW0[:D], D, W), _pad_w(dW0[D:], W, W), _pad_b(db0, W),
                    _pad_w(dW1, W, W), _pad_b(db1, W),
                    _pad_w(dW2, W, W), _pad_b(db2, W),
                    _pad_w(dW3, W, W), _pad_b(db3, W),
                    _pad_w(dW4, W, W), _pad_b(db4, W))
    return (x_r[:, :XDIM], sim[:E])


# speedup vs baseline: 1.8163x; 1.8163x over previous
import functools

import jax
import jax.numpy as jnp
from jax import lax
from jax.experimental import pallas as pl
from jax.experimental.pallas import tpu as pltpu
from jax.experimental.pallas import tpu_sc as plsc

N = 10000
D = 128
E = 320000
H = 20
OH = 20
XDIM = 20

WS = 24
W = 32
NC = 2
NS = 16
NW = NC * NS
SEG_CH = 512
SEG_CPW = 20
DOT_CH = 512
DOT_CPW = 20
EPW = SEG_CPW * SEG_CH
E_PAD = NW * EPW
N_ACC = N + 112
ZR = N_ACC // NS

BN = 1000
f32 = jnp.float32


def _sc_mesh():
    return plsc.VectorSubcoreMesh(core_axis_name="c", subcore_axis_name="s",
                                  num_cores=NC, num_subcores=NS)


_SC_PARAMS = pltpu.CompilerParams(use_tc_tiling_on_sc=False,
                                  needs_layout_passes=False)


def _segsum_body(p_hbm, src_hbm, dst_hbm, zero_hbm, out_hbm,
                 src_v, dst_v, rows_v, acc_sh, sem):
    c = lax.axis_index("c")
    s = lax.axis_index("s")
    w = c * NS + s
    pltpu.sync_copy(zero_hbm, acc_sh.at[pl.ds(s * ZR, ZR)])
    pltpu.sync_copy(src_hbm.at[w], src_v)
    pltpu.sync_copy(dst_hbm.at[w], dst_v)
    plsc.subcore_barrier()

    def chunk(j, carry):
        pltpu.async_copy(p_hbm.at[src_v.at[j]], rows_v, sem).wait()
        pltpu.sync_copy(rows_v, acc_sh.at[dst_v.at[j]], add=True)
        return carry

    lax.fori_loop(0, SEG_CPW, chunk, 0)
    plsc.subcore_barrier()
    pltpu.sync_copy(acc_sh.at[pl.ds(s * ZR, ZR)],
                    out_hbm.at[c].at[pl.ds(s * ZR, ZR)])


@functools.cache
def _segsum_kernel():
    return pl.kernel(
        _segsum_body,
        out_type=jax.ShapeDtypeStruct((NC, N_ACC, WS), f32),
        mesh=_sc_mesh(),
        scratch_types=[
            pltpu.VMEM((SEG_CPW, SEG_CH), jnp.int32),
            pltpu.VMEM((SEG_CPW, SEG_CH), jnp.int32),
            pltpu.VMEM((SEG_CH, WS), f32),
            pltpu.VMEM_SHARED((N_ACC, WS), f32),
            pltpu.SemaphoreType.DMA,
        ],
        compiler_params=_SC_PARAMS,
    )


def _segsum(p, src, dst, zero):
    return _segsum_kernel()(p, src, dst, zero)


def _edgedot_body(zz_hbm, idx_hbm, out_hbm, idx_v, rows_v, sim_v, sem):
    c = lax.axis_index("c")
    s = lax.axis_index("s")
    w = c * NS + s
    pltpu.sync_copy(idx_hbm.at[w], idx_v)
    lane = lax.iota(jnp.int32, 16)

    def chunk(j, carry):
        pltpu.async_copy(zz_hbm.at[idx_v.at[j]], rows_v, sem).wait()
        for g in range(DOT_CH // 16):
            rows = lane + (g * 16)
            acc = jnp.zeros((16,), f32)
            for f in range(OH):
                col = jnp.full((16,), f, jnp.int32)
                acc = acc + (plsc.load_gather(rows_v, (rows, col))
                             * plsc.load_gather(rows_v, (rows + DOT_CH, col)))
            sim_v[pl.ds(g * 16, 16)] = 1.0 / (1.0 + jnp.exp(-acc))
        pltpu.sync_copy(sim_v, out_hbm.at[pl.ds((j * NW + w) * DOT_CH, DOT_CH)])
        return carry

    lax.fori_loop(0, DOT_CPW, chunk, 0)


@functools.cache
def _edgedot_kernel():
    return pl.kernel(
        _edgedot_body,
        out_type=jax.ShapeDtypeStruct((E_PAD,), f32),
        mesh=_sc_mesh(),
        scratch_types=[
            pltpu.VMEM((DOT_CPW, 2 * DOT_CH), jnp.int32),
            pltpu.VMEM((2 * DOT_CH, WS), f32),
            pltpu.VMEM((DOT_CH,), f32),
            pltpu.SemaphoreType.DMA,
        ],
        compiler_params=_SC_PARAMS,
    )


def _edgedot(zz, idx):
    return _edgedot_kernel()(zz, idx)


def _ones_col():
    col = lax.broadcasted_iota(jnp.int32, (1, WS), 1)
    return jnp.where(col == H, 1.0, 0.0).astype(f32)


def _prep0_body(z_ref, wl_ref, wr_ref, b_ref, p_ref, r_ref):
    zb = z_ref[...]
    p_ref[...] = jnp.dot(zb, wl_ref[...], preferred_element_type=f32) + _ones_col()
    r_ref[...] = jnp.dot(zb, wr_ref[...], preferred_element_type=f32) + b_ref[...]


def _combine(pa, pb, r):
    ssum = pa + pb
    col = lax.broadcasted_iota(jnp.int32, (1, WS), 1)
    cnt = jnp.sum(jnp.where(col == H, ssum, 0.0), axis=1, keepdims=True)
    mean = ssum / jnp.maximum(cnt, 1.0)
    return jnp.maximum(mean + r, 0.0)


def _comb_prep_body(pa_ref, pb_ref, r_ref, wl_ref, wr_ref, b_ref, p_ref, rn_ref):
    h = _combine(pa_ref[...], pb_ref[...], r_ref[...])
    p_ref[...] = jnp.dot(h, wl_ref[...], preferred_element_type=f32) + _ones_col()
    rn_ref[...] = jnp.dot(h, wr_ref[...], preferred_element_type=f32) + b_ref[...]


def _zz_body(pa_ref, pb_ref, r_ref, lw_ref, lb_ref, zz_ref):
    h = _combine(pa_ref[...], pb_ref[...], r_ref[...])
    zz_ref[...] = jnp.dot(h, lw_ref[...], preferred_element_type=f32) + lb_ref[...]


def _dec_body(z_ref, zz_ref, w0a_ref, w0b_ref, b0_ref, w1_ref, b1_ref,
              w2_ref, b2_ref, w3_ref, b3_ref, w4_ref, b4_ref, out_ref):
    x = jnp.maximum(jnp.dot(z_ref[...], w0a_ref[...], preferred_element_type=f32)
                    + jnp.dot(zz_ref[...], w0b_ref[...], preferred_element_type=f32)
                    + b0_ref[...], 0.0)
    for wr, br in ((w1_ref, b1_ref), (w2_ref, b2_ref), (w3_ref, b3_ref)):
        x = jnp.maximum(jnp.dot(x, wr[...], preferred_element_type=f32) + br[...], 0.0)
    lg = jnp.dot(x, w4_ref[...], preferred_element_type=f32) + b4_ref[...]
    col = lax.broadcasted_iota(jnp.int32, (1, W), 1)
    neg = jnp.where(col < XDIM, lg, -1e30)
    m = jnp.max(neg, axis=1, keepdims=True)
    ex = jnp.where(col < XDIM, jnp.exp(neg - m), 0.0)
    out_ref[...] = (neg - m) - jnp.log(jnp.sum(ex, axis=1, keepdims=True))


def _full(shape):
    return pl.BlockSpec(shape, lambda i: (0, 0))


def _rows(width):
    return pl.BlockSpec((BN, width), lambda i: (i, 0))


_GRID = (N // BN,)


def _call_prep0(z, wl, wr, b):
    return pl.pallas_call(
        _prep0_body, grid=_GRID,
        in_specs=[_rows(D), _full((D, WS)), _full((D, WS)), _full((1, WS))],
        out_specs=[_rows(WS), _rows(WS)],
        out_shape=[jax.ShapeDtypeStruct((N, WS), f32)] * 2,
    )(z, wl, wr, b)


def _call_comb_prep(pa, pb, r, wl, wr, b):
    return pl.pallas_call(
        _comb_prep_body, grid=_GRID,
        in_specs=[_rows(WS), _rows(WS), _rows(WS),
                  _full((WS, WS)), _full((WS, WS)), _full((1, WS))],
        out_specs=[_rows(WS), _rows(WS)],
        out_shape=[jax.ShapeDtypeStruct((N, WS), f32)] * 2,
    )(pa, pb, r, wl, wr, b)


def _call_zz(pa, pb, r, lw, lb):
    return pl.pallas_call(
        _zz_body, grid=_GRID,
        in_specs=[_rows(WS), _rows(WS), _rows(WS), _full((WS, WS)), _full((1, WS))],
        out_specs=_rows(WS),
        out_shape=jax.ShapeDtypeStruct((N, WS), f32),
    )(pa, pb, r, lw, lb)


def _call_dec(z, zz, w0a, w0b, b0, w1, b1, w2, b2, w3, b3, w4, b4):
    return pl.pallas_call(
        _dec_body, grid=_GRID,
        in_specs=[_rows(D), _rows(WS),
                  _full((D, W)), _full((WS, W)), _full((1, W)),
                  _full((W, W)), _full((1, W)),
                  _full((W, W)), _full((1, W)),
                  _full((W, W)), _full((1, W)),
                  _full((W, W)), _full((1, W))],
        out_specs=_rows(W),
        out_shape=jax.ShapeDtypeStruct((N, W), f32),
    )(z, zz, w0a, w0b, b0, w1, b1, w2, b2, w3, b3, w4, b4)


def _pad_w(w, rows, cols):
    return jnp.zeros((rows, cols), f32).at[:w.shape[0], :w.shape[1]].set(w)


def _pad_b(b, cols):
    return jnp.zeros((1, cols), f32).at[0, :b.shape[0]].set(b)


def _pad_flat(x, fill):
    pad = E_PAD - E
    return jnp.concatenate([x, jnp.full((pad,), 0, jnp.int32) + fill])


def _prep_seg_edges(src, dst):
    pad = E_PAD - E
    fill = N + jnp.arange(pad, dtype=jnp.int32) % (N_ACC - N)
    srcp = _pad_flat(src, 0)
    dstp = jnp.concatenate([dst, fill])
    srcp = srcp.reshape(SEG_CPW, NW, SEG_CH).transpose(1, 0, 2)
    dstp = dstp.reshape(SEG_CPW, NW, SEG_CH).transpose(1, 0, 2)
    return srcp, dstp


def _prep_dot_edges(src, dst):
    srcp = _pad_flat(src, 0).reshape(DOT_CPW, NW, DOT_CH)
    dstp = _pad_flat(dst, 0).reshape(DOT_CPW, NW, DOT_CH)
    comb = jnp.concatenate([srcp, dstp], axis=2)
    return comb.transpose(1, 0, 2)


def kernel(z, edge_index, backbones, Wl0, Wr0, b0, Wl1, Wr1, b1, Wl2, Wr2, b2,
           linW, linB, dW0, db0, dW1, db1, dW2, db2, dW3, db3, dW4, db4):
    sb, db = _prep_seg_edges(backbones[0], backbones[1])
    ed = _prep_dot_edges(edge_index[0], edge_index[1])
    zero_rows = jnp.zeros((ZR, WS), f32)

    p, r = _call_prep0(z, _pad_w(Wl0, D, WS), _pad_w(Wr0, D, WS), _pad_b(b0, WS))
    part = _segsum(p, sb, db, zero_rows)
    p, r = _call_comb_prep(part[0], part[1], r,
                           _pad_w(Wl1, WS, WS), _pad_w(Wr1, WS, WS), _pad_b(b1, WS))
    part = _segsum(p, sb, db, zero_rows)
    p, r = _call_comb_prep(part[0], part[1], r,
                           _pad_w(Wl2, WS, WS), _pad_w(Wr2, WS, WS), _pad_b(b2, WS))
    part = _segsum(p, sb, db, zero_rows)
    zz = _call_zz(part[0], part[1], r, _pad_w(linW, WS, WS), _pad_b(linB, WS))

    sim = _edgedot(zz, ed)
    x_r = _call_dec(z, zz,
                    _pad_w(dW0[:D], D, W), _pad_w(dW0[D:], WS, W), _pad_b(db0, W),
                    _pad_w(dW1, W, W), _pad_b(db1, W),
                    _pad_w(dW2, W, W), _pad_b(db2, W),
                    _pad_w(dW3, W, W), _pad_b(db3, W),
                    _pad_w(dW4, W, W), _pad_b(db4, W))
    return (x_r[:, :XDIM], sim[:E])


# speedup vs baseline: 1.8683x; 1.0287x over previous
import functools

import jax
import jax.numpy as jnp
from jax import lax
from jax.experimental import pallas as pl
from jax.experimental.pallas import tpu as pltpu
from jax.experimental.pallas import tpu_sc as plsc

N = 10000
D = 128
E = 320000
H = 20
OH = 20
XDIM = 20

WS = 24
W = 32
NC = 2
NS = 16
NW = NC * NS
CH = 512
CT = 640
E_PAD = CT * CH
CPW0 = 26
CPW1 = 14
CPW_MAX = 26
N_ACC = N + 112
ZR = N_ACC // NS

BN = 1000
f32 = jnp.float32


def _sc_mesh():
    return plsc.VectorSubcoreMesh(core_axis_name="c", subcore_axis_name="s",
                                  num_cores=NC, num_subcores=NS)


_SC_PARAMS = pltpu.CompilerParams(use_tc_tiling_on_sc=False,
                                  needs_layout_passes=False)


def _segsum_body(p_hbm, src_hbm, dst_hbm, zero_hbm, out_hbm,
                 src_v, dst_v, rows_v, acc_sh, sem):
    c = lax.axis_index("c")
    s = lax.axis_index("s")
    w = c * NS + s
    pltpu.sync_copy(zero_hbm, acc_sh.at[pl.ds(s * ZR, ZR)])
    pltpu.sync_copy(src_hbm.at[w], src_v)
    pltpu.sync_copy(dst_hbm.at[w], dst_v)
    plsc.subcore_barrier()

    def chunk(j, carry):
        pltpu.async_copy(p_hbm.at[src_v.at[j]], rows_v, sem).wait()
        pltpu.sync_copy(rows_v, acc_sh.at[dst_v.at[j]], add=True)
        return carry

    lax.fori_loop(0, jnp.where(c == 0, CPW0, CPW1), chunk, 0)
    plsc.subcore_barrier()
    pltpu.sync_copy(acc_sh.at[pl.ds(s * ZR, ZR)],
                    out_hbm.at[c].at[pl.ds(s * ZR, ZR)])


@functools.cache
def _segsum_kernel():
    return pl.kernel(
        _segsum_body,
        out_type=jax.ShapeDtypeStruct((NC, N_ACC, WS), f32),
        mesh=_sc_mesh(),
        scratch_types=[
            pltpu.VMEM((CPW_MAX, CH), jnp.int32),
            pltpu.VMEM((CPW_MAX, CH), jnp.int32),
            pltpu.VMEM((CH, WS), f32),
            pltpu.VMEM_SHARED((N_ACC, WS), f32),
            pltpu.SemaphoreType.DMA,
        ],
        compiler_params=_SC_PARAMS,
    )


def _segsum(p, src, dst, zero):
    return _segsum_kernel()(p, src, dst, zero)


def _edgedot_body(zz_hbm, idx_hbm, out_hbm, idx_v, rows_v, sim_v, sem):
    c = lax.axis_index("c")
    s = lax.axis_index("s")
    w = c * NS + s
    pltpu.sync_copy(idx_hbm.at[w], idx_v)
    lane = lax.iota(jnp.int32, 16)

    def chunk(j, carry):
        pltpu.async_copy(zz_hbm.at[idx_v.at[j]], rows_v, sem).wait()
        for g in range(CH // 16):
            rows = lane + (g * 16)
            acc = jnp.zeros((16,), f32)
            for f in range(OH):
                col = jnp.full((16,), f, jnp.int32)
                acc = acc + (plsc.load_gather(rows_v, (rows, col))
                             * plsc.load_gather(rows_v, (rows + CH, col)))
            sim_v[pl.ds(g * 16, 16)] = 1.0 / (1.0 + jnp.exp(-acc))
        k = jnp.where(c == 0, j * NS + s, NS * CPW0 + j * NS + s)
        pltpu.sync_copy(sim_v, out_hbm.at[pl.ds(k * CH, CH)])
        return carry

    lax.fori_loop(0, jnp.where(c == 0, CPW0, CPW1), chunk, 0)


@functools.cache
def _edgedot_kernel():
    return pl.kernel(
        _edgedot_body,
        out_type=jax.ShapeDtypeStruct((E_PAD,), f32),
        mesh=_sc_mesh(),
        scratch_types=[
            pltpu.VMEM((CPW_MAX, 2 * CH), jnp.int32),
            pltpu.VMEM((2 * CH, WS), f32),
            pltpu.VMEM((CH,), f32),
            pltpu.SemaphoreType.DMA,
        ],
        compiler_params=_SC_PARAMS,
    )


def _edgedot(zz, idx):
    return _edgedot_kernel()(zz, idx)


def _ones_col():
    col = lax.broadcasted_iota(jnp.int32, (1, WS), 1)
    return jnp.where(col == H, 1.0, 0.0).astype(f32)


def _prep0_body(z_ref, wl_ref, wr_ref, b_ref, p_ref, r_ref):
    zb = z_ref[...]
    p_ref[...] = jnp.dot(zb, wl_ref[...], preferred_element_type=f32) + _ones_col()
    r_ref[...] = jnp.dot(zb, wr_ref[...], preferred_element_type=f32) + b_ref[...]


def _combine(pa, pb, r):
    ssum = pa + pb
    col = lax.broadcasted_iota(jnp.int32, (1, WS), 1)
    cnt = jnp.sum(jnp.where(col == H, ssum, 0.0), axis=1, keepdims=True)
    mean = ssum / jnp.maximum(cnt, 1.0)
    return jnp.maximum(mean + r, 0.0)


def _comb_prep_body(pa_ref, pb_ref, r_ref, wl_ref, wr_ref, b_ref, p_ref, rn_ref):
    h = _combine(pa_ref[...], pb_ref[...], r_ref[...])
    p_ref[...] = jnp.dot(h, wl_ref[...], preferred_element_type=f32) + _ones_col()
    rn_ref[...] = jnp.dot(h, wr_ref[...], preferred_element_type=f32) + b_ref[...]


def _zz_body(pa_ref, pb_ref, r_ref, lw_ref, lb_ref, zz_ref):
    h = _combine(pa_ref[...], pb_ref[...], r_ref[...])
    zz_ref[...] = jnp.dot(h, lw_ref[...], preferred_element_type=f32) + lb_ref[...]


def _dec_body(z_ref, zz_ref, w0a_ref, w0b_ref, b0_ref, w1_ref, b1_ref,
              w2_ref, b2_ref, w3_ref, b3_ref, w4_ref, b4_ref, out_ref):
    x = jnp.maximum(jnp.dot(z_ref[...], w0a_ref[...], preferred_element_type=f32)
                    + jnp.dot(zz_ref[...], w0b_ref[...], preferred_element_type=f32)
                    + b0_ref[...], 0.0)
    for wr, br in ((w1_ref, b1_ref), (w2_ref, b2_ref), (w3_ref, b3_ref)):
        x = jnp.maximum(jnp.dot(x, wr[...], preferred_element_type=f32) + br[...], 0.0)
    lg = jnp.dot(x, w4_ref[...], preferred_element_type=f32) + b4_ref[...]
    col = lax.broadcasted_iota(jnp.int32, (1, W), 1)
    neg = jnp.where(col < XDIM, lg, -1e30)
    m = jnp.max(neg, axis=1, keepdims=True)
    ex = jnp.where(col < XDIM, jnp.exp(neg - m), 0.0)
    out_ref[...] = (neg - m) - jnp.log(jnp.sum(ex, axis=1, keepdims=True))


def _full(shape):
    return pl.BlockSpec(shape, lambda i: (0, 0))


def _rows(width):
    return pl.BlockSpec((BN, width), lambda i: (i, 0))


_GRID = (N // BN,)


def _call_prep0(z, wl, wr, b):
    return pl.pallas_call(
        _prep0_body, grid=_GRID,
        in_specs=[_rows(D), _full((D, WS)), _full((D, WS)), _full((1, WS))],
        out_specs=[_rows(WS), _rows(WS)],
        out_shape=[jax.ShapeDtypeStruct((N, WS), f32)] * 2,
    )(z, wl, wr, b)


def _call_comb_prep(pa, pb, r, wl, wr, b):
    return pl.pallas_call(
        _comb_prep_body, grid=_GRID,
        in_specs=[_rows(WS), _rows(WS), _rows(WS),
                  _full((WS, WS)), _full((WS, WS)), _full((1, WS))],
        out_specs=[_rows(WS), _rows(WS)],
        out_shape=[jax.ShapeDtypeStruct((N, WS), f32)] * 2,
    )(pa, pb, r, wl, wr, b)


def _call_zz(pa, pb, r, lw, lb):
    return pl.pallas_call(
        _zz_body, grid=_GRID,
        in_specs=[_rows(WS), _rows(WS), _rows(WS), _full((WS, WS)), _full((1, WS))],
        out_specs=_rows(WS),
        out_shape=jax.ShapeDtypeStruct((N, WS), f32),
    )(pa, pb, r, lw, lb)


def _call_dec(z, zz, w0a, w0b, b0, w1, b1, w2, b2, w3, b3, w4, b4):
    return pl.pallas_call(
        _dec_body, grid=_GRID,
        in_specs=[_rows(D), _rows(WS),
                  _full((D, W)), _full((WS, W)), _full((1, W)),
                  _full((W, W)), _full((1, W)),
                  _full((W, W)), _full((1, W)),
                  _full((W, W)), _full((1, W)),
                  _full((W, W)), _full((1, W))],
        out_specs=_rows(W),
        out_shape=jax.ShapeDtypeStruct((N, W), f32),
    )(z, zz, w0a, w0b, b0, w1, b1, w2, b2, w3, b3, w4, b4)


def _pad_w(w, rows, cols):
    return jnp.zeros((rows, cols), f32).at[:w.shape[0], :w.shape[1]].set(w)


def _pad_b(b, cols):
    return jnp.zeros((1, cols), f32).at[0, :b.shape[0]].set(b)


def _pad_flat(x, fill):
    pad = E_PAD - E
    return jnp.concatenate([x, jnp.full((pad,), 0, jnp.int32) + fill])


def _split_chunks(flat, ch):
    chunks = flat.reshape(CT, ch)
    n0 = NS * CPW0
    c0 = chunks[:n0].reshape(CPW0, NS, ch).transpose(1, 0, 2)
    c1 = chunks[n0:].reshape(CPW1, NS, ch).transpose(1, 0, 2)
    c1 = jnp.concatenate(
        [c1, jnp.zeros((NS, CPW_MAX - CPW1, ch), jnp.int32)], axis=1)
    return jnp.concatenate([c0, c1], axis=0)


def _prep_seg_edges(src, dst):
    pad = E_PAD - E
    fill = N + jnp.arange(pad, dtype=jnp.int32) % (N_ACC - N)
    return (_split_chunks(_pad_flat(src, 0), CH),
            _split_chunks(jnp.concatenate([dst, fill]), CH))


def _prep_dot_edges(src, dst):
    srcp = _pad_flat(src, 0).reshape(CT, CH)
    dstp = _pad_flat(dst, 0).reshape(CT, CH)
    comb = jnp.concatenate([srcp, dstp], axis=1).reshape(-1)
    return _split_chunks(comb, 2 * CH)


def kernel(z, edge_index, backbones, Wl0, Wr0, b0, Wl1, Wr1, b1, Wl2, Wr2, b2,
           linW, linB, dW0, db0, dW1, db1, dW2, db2, dW3, db3, dW4, db4):
    sb, db = _prep_seg_edges(backbones[0], backbones[1])
    ed = _prep_dot_edges(edge_index[0], edge_index[1])
    zero_rows = jnp.zeros((ZR, WS), f32)

    p, r = _call_prep0(z, _pad_w(Wl0, D, WS), _pad_w(Wr0, D, WS), _pad_b(b0, WS))
    part = _segsum(p, sb, db, zero_rows)
    p, r = _call_comb_prep(part[0], part[1], r,
                           _pad_w(Wl1, WS, WS), _pad_w(Wr1, WS, WS), _pad_b(b1, WS))
    part = _segsum(p, sb, db, zero_rows)
    p, r = _call_comb_prep(part[0], part[1], r,
                           _pad_w(Wl2, WS, WS), _pad_w(Wr2, WS, WS), _pad_b(b2, WS))
    part = _segsum(p, sb, db, zero_rows)
    zz = _call_zz(part[0], part[1], r, _pad_w(linW, WS, WS), _pad_b(linB, WS))

    sim = _edgedot(zz, ed)
    x_r = _call_dec(z, zz,
                    _pad_w(dW0[:D], D, W), _pad_w(dW0[D:], WS, W), _pad_b(db0, W),
                    _pad_w(dW1, W, W), _pad_b(db1, W),
                    _pad_w(dW2, W, W), _pad_b(db2, W),
                    _pad_w(dW3, W, W), _pad_b(db3, W),
                    _pad_w(dW4, W, W), _pad_b(db4, W))
    return (x_r[:, :XDIM], sim[:E])


# speedup vs baseline: 1.8748x; 1.0035x over previous
import functools

import jax
import jax.numpy as jnp
from jax import lax
from jax.experimental import pallas as pl
from jax.experimental.pallas import tpu as pltpu
from jax.experimental.pallas import tpu_sc as plsc

N = 10000
D = 128
E = 320000
H = 20
OH = 20
XDIM = 20

WS = 24
W = 32
NC = 2
NS = 16
NW = NC * NS
CH = 512
CT = 640
E_PAD = CT * CH
CPW0 = 26
CPW1 = 14
CPW_MAX = 26
N_ACC = N + 112
ZR = N_ACC // NS

BN = 1000
f32 = jnp.float32


def _sc_mesh():
    return plsc.VectorSubcoreMesh(core_axis_name="c", subcore_axis_name="s",
                                  num_cores=NC, num_subcores=NS)


_SC_PARAMS = pltpu.CompilerParams(use_tc_tiling_on_sc=False,
                                  needs_layout_passes=False)


def _segsum_body(p_hbm, src_hbm, dst_hbm, zero_hbm, out_hbm,
                 src_v, dst_v, rows_v, acc_sh, sem):
    c = lax.axis_index("c")
    s = lax.axis_index("s")
    w = c * NS + s
    pltpu.sync_copy(zero_hbm, acc_sh.at[pl.ds(s * ZR, ZR)])
    pltpu.sync_copy(src_hbm.at[w], src_v)
    pltpu.sync_copy(dst_hbm.at[w], dst_v)
    plsc.subcore_barrier()

    def chunk(j, carry):
        pltpu.async_copy(p_hbm.at[src_v.at[j]], rows_v, sem).wait()
        pltpu.sync_copy(rows_v, acc_sh.at[dst_v.at[j]], add=True)
        return carry

    @pl.when(c == 0)
    def _():
        lax.fori_loop(0, CPW0, chunk, 0, unroll=False)

    @pl.when(c != 0)
    def _():
        lax.fori_loop(0, CPW1, chunk, 0, unroll=False)

    plsc.subcore_barrier()
    pltpu.sync_copy(acc_sh.at[pl.ds(s * ZR, ZR)],
                    out_hbm.at[c].at[pl.ds(s * ZR, ZR)])


@functools.cache
def _segsum_kernel():
    return pl.kernel(
        _segsum_body,
        out_type=jax.ShapeDtypeStruct((NC, N_ACC, WS), f32),
        mesh=_sc_mesh(),
        scratch_types=[
            pltpu.VMEM((CPW_MAX, CH), jnp.int32),
            pltpu.VMEM((CPW_MAX, CH), jnp.int32),
            pltpu.VMEM((CH, WS), f32),
            pltpu.VMEM_SHARED((N_ACC, WS), f32),
            pltpu.SemaphoreType.DMA,
        ],
        compiler_params=_SC_PARAMS,
    )


def _segsum(p, src, dst, zero):
    return _segsum_kernel()(p, src, dst, zero)


def _edgedot_body(zz_hbm, idx_hbm, out_hbm, idx_v, rows_v, sim_v, sem):
    c = lax.axis_index("c")
    s = lax.axis_index("s")
    w = c * NS + s
    pltpu.sync_copy(idx_hbm.at[w], idx_v)
    lane = lax.iota(jnp.int32, 16)

    def chunk(j, carry):
        pltpu.async_copy(zz_hbm.at[idx_v.at[j]], rows_v, sem).wait()
        for g in range(CH // 16):
            rows = lane + (g * 16)
            acc = jnp.zeros((16,), f32)
            for f in range(OH):
                col = jnp.full((16,), f, jnp.int32)
                acc = acc + (plsc.load_gather(rows_v, (rows, col))
                             * plsc.load_gather(rows_v, (rows + CH, col)))
            sim_v[pl.ds(g * 16, 16)] = 1.0 / (1.0 + jnp.exp(-acc))
        k = jnp.where(c == 0, j * NS + s, NS * CPW0 + j * NS + s)
        pltpu.sync_copy(sim_v, out_hbm.at[pl.ds(k * CH, CH)])
        return carry

    @pl.when(c == 0)
    def _():
        lax.fori_loop(0, CPW0, chunk, 0, unroll=False)

    @pl.when(c != 0)
    def _():
        lax.fori_loop(0, CPW1, chunk, 0, unroll=False)


@functools.cache
def _edgedot_kernel():
    return pl.kernel(
        _edgedot_body,
        out_type=jax.ShapeDtypeStruct((E_PAD,), f32),
        mesh=_sc_mesh(),
        scratch_types=[
            pltpu.VMEM((CPW_MAX, 2 * CH), jnp.int32),
            pltpu.VMEM((2 * CH, WS), f32),
            pltpu.VMEM((CH,), f32),
            pltpu.SemaphoreType.DMA,
        ],
        compiler_params=_SC_PARAMS,
    )


def _edgedot(zz, idx):
    return _edgedot_kernel()(zz, idx)


def _ones_col():
    col = lax.broadcasted_iota(jnp.int32, (1, WS), 1)
    return jnp.where(col == H, 1.0, 0.0).astype(f32)


def _prep0_body(z_ref, wl_ref, wr_ref, b_ref, p_ref, r_ref):
    zb = z_ref[...]
    p_ref[...] = jnp.dot(zb, wl_ref[...], preferred_element_type=f32) + _ones_col()
    r_ref[...] = jnp.dot(zb, wr_ref[...], preferred_element_type=f32) + b_ref[...]


def _combine(pa, pb, r):
    ssum = pa + pb
    col = lax.broadcasted_iota(jnp.int32, (1, WS), 1)
    cnt = jnp.sum(jnp.where(col == H, ssum, 0.0), axis=1, keepdims=True)
    mean = ssum / jnp.maximum(cnt, 1.0)
    return jnp.maximum(mean + r, 0.0)


def _comb_prep_body(pa_ref, pb_ref, r_ref, wl_ref, wr_ref, b_ref, p_ref, rn_ref):
    h = _combine(pa_ref[...], pb_ref[...], r_ref[...])
    p_ref[...] = jnp.dot(h, wl_ref[...], preferred_element_type=f32) + _ones_col()
    rn_ref[...] = jnp.dot(h, wr_ref[...], preferred_element_type=f32) + b_ref[...]


def _zz_body(pa_ref, pb_ref, r_ref, lw_ref, lb_ref, zz_ref):
    h = _combine(pa_ref[...], pb_ref[...], r_ref[...])
    zz_ref[...] = jnp.dot(h, lw_ref[...], preferred_element_type=f32) + lb_ref[...]


def _dec_body(z_ref, zz_ref, w0a_ref, w0b_ref, b0_ref, w1_ref, b1_ref,
              w2_ref, b2_ref, w3_ref, b3_ref, w4_ref, b4_ref, out_ref):
    x = jnp.maximum(jnp.dot(z_ref[...], w0a_ref[...], preferred_element_type=f32)
                    + jnp.dot(zz_ref[...], w0b_ref[...], preferred_element_type=f32)
                    + b0_ref[...], 0.0)
    for wr, br in ((w1_ref, b1_ref), (w2_ref, b2_ref), (w3_ref, b3_ref)):
        x = jnp.maximum(jnp.dot(x, wr[...], preferred_element_type=f32) + br[...], 0.0)
    lg = jnp.dot(x, w4_ref[...], preferred_element_type=f32) + b4_ref[...]
    col = lax.broadcasted_iota(jnp.int32, (1, W), 1)
    neg = jnp.where(col < XDIM, lg, -1e30)
    m = jnp.max(neg, axis=1, keepdims=True)
    ex = jnp.where(col < XDIM, jnp.exp(neg - m), 0.0)
    out_ref[...] = (neg - m) - jnp.log(jnp.sum(ex, axis=1, keepdims=True))


def _full(shape):
    return pl.BlockSpec(shape, lambda i: (0, 0))


def _rows(width):
    return pl.BlockSpec((BN, width), lambda i: (i, 0))


_GRID = (N // BN,)


def _call_prep0(z, wl, wr, b):
    return pl.pallas_call(
        _prep0_body, grid=_GRID,
        in_specs=[_rows(D), _full((D, WS)), _full((D, WS)), _full((1, WS))],
        out_specs=[_rows(WS), _rows(WS)],
        out_shape=[jax.ShapeDtypeStruct((N, WS), f32)] * 2,
    )(z, wl, wr, b)


def _call_comb_prep(pa, pb, r, wl, wr, b):
    return pl.pallas_call(
        _comb_prep_body, grid=_GRID,
        in_specs=[_rows(WS), _rows(WS), _rows(WS),
                  _full((WS, WS)), _full((WS, WS)), _full((1, WS))],
        out_specs=[_rows(WS), _rows(WS)],
        out_shape=[jax.ShapeDtypeStruct((N, WS), f32)] * 2,
    )(pa, pb, r, wl, wr, b)


def _call_zz(pa, pb, r, lw, lb):
    return pl.pallas_call(
        _zz_body, grid=_GRID,
        in_specs=[_rows(WS), _rows(WS), _rows(WS), _full((WS, WS)), _full((1, WS))],
        out_specs=_rows(WS),
        out_shape=jax.ShapeDtypeStruct((N, WS), f32),
    )(pa, pb, r, lw, lb)


def _call_dec(z, zz, w0a, w0b, b0, w1, b1, w2, b2, w3, b3, w4, b4):
    return pl.pallas_call(
        _dec_body, grid=_GRID,
        in_specs=[_rows(D), _rows(WS),
                  _full((D, W)), _full((WS, W)), _full((1, W)),
                  _full((W, W)), _full((1, W)),
                  _full((W, W)), _full((1, W)),
                  _full((W, W)), _full((1, W)),
                  _full((W, W)), _full((1, W))],
        out_specs=_rows(W),
        out_shape=jax.ShapeDtypeStruct((N, W), f32),
    )(z, zz, w0a, w0b, b0, w1, b1, w2, b2, w3, b3, w4, b4)


def _pad_w(w, rows, cols):
    return jnp.zeros((rows, cols), f32).at[:w.shape[0], :w.shape[1]].set(w)


def _pad_b(b, cols):
    return jnp.zeros((1, cols), f32).at[0, :b.shape[0]].set(b)


def _pad_flat(x, fill):
    pad = E_PAD - E
    return jnp.concatenate([x, jnp.full((pad,), 0, jnp.int32) + fill])


def _split_chunks(flat, ch):
    chunks = flat.reshape(CT, ch)
    n0 = NS * CPW0
    c0 = chunks[:n0].reshape(CPW0, NS, ch).transpose(1, 0, 2)
    c1 = chunks[n0:].reshape(CPW1, NS, ch).transpose(1, 0, 2)
    c1 = jnp.concatenate(
        [c1, jnp.zeros((NS, CPW_MAX - CPW1, ch), jnp.int32)], axis=1)
    return jnp.concatenate([c0, c1], axis=0)


def _prep_seg_edges(src, dst):
    pad = E_PAD - E
    fill = N + jnp.arange(pad, dtype=jnp.int32) % (N_ACC - N)
    return (_split_chunks(_pad_flat(src, 0), CH),
            _split_chunks(jnp.concatenate([dst, fill]), CH))


def _prep_dot_edges(src, dst):
    srcp = _pad_flat(src, 0).reshape(CT, CH)
    dstp = _pad_flat(dst, 0).reshape(CT, CH)
    comb = jnp.concatenate([srcp, dstp], axis=1).reshape(-1)
    return _split_chunks(comb, 2 * CH)


def kernel(z, edge_index, backbones, Wl0, Wr0, b0, Wl1, Wr1, b1, Wl2, Wr2, b2,
           linW, linB, dW0, db0, dW1, db1, dW2, db2, dW3, db3, dW4, db4):
    sb, db = _prep_seg_edges(backbones[0], backbones[1])
    ed = _prep_dot_edges(edge_index[0], edge_index[1])
    zero_rows = jnp.zeros((ZR, WS), f32)

    p, r = _call_prep0(z, _pad_w(Wl0, D, WS), _pad_w(Wr0, D, WS), _pad_b(b0, WS))
    part = _segsum(p, sb, db, zero_rows)
    p, r = _call_comb_prep(part[0], part[1], r,
                           _pad_w(Wl1, WS, WS), _pad_w(Wr1, WS, WS), _pad_b(b1, WS))
    part = _segsum(p, sb, db, zero_rows)
    p, r = _call_comb_prep(part[0], part[1], r,
                           _pad_w(Wl2, WS, WS), _pad_w(Wr2, WS, WS), _pad_b(b2, WS))
    part = _segsum(p, sb, db, zero_rows)
    zz = _call_zz(part[0], part[1], r, _pad_w(linW, WS, WS), _pad_b(linB, WS))

    sim = _edgedot(zz, ed)
    x_r = _call_dec(z, zz,
                    _pad_w(dW0[:D], D, W), _pad_w(dW0[D:], WS, W), _pad_b(db0, W),
                    _pad_w(dW1, W, W), _pad_b(db1, W),
                    _pad_w(dW2, W, W), _pad_b(db2, W),
                    _pad_w(dW3, W, W), _pad_b(db3, W),
                    _pad_w(dW4, W, W), _pad_b(db4, W))
    return (x_r[:, :XDIM], sim[:E])


# speedup vs baseline: 1.9123x; 1.0200x over previous
import functools

import jax
import jax.numpy as jnp
from jax import lax
from jax.experimental import pallas as pl
from jax.experimental.pallas import tpu as pltpu
from jax.experimental.pallas import tpu_sc as plsc

N = 10000
D = 128
E = 320000
H = 20
OH = 20
XDIM = 20

WS = 24
W = 32
NC = 2
NS = 16
NW = NC * NS
CH = 1024
CT = 320
E_PAD = CT * CH
CPW0 = 13
CPW1 = 7
CPW_MAX = 13
N_ACC = N + 112
ZR = N_ACC // NS

BN = 1000
f32 = jnp.float32


def _sc_mesh():
    return plsc.VectorSubcoreMesh(core_axis_name="c", subcore_axis_name="s",
                                  num_cores=NC, num_subcores=NS)


_SC_PARAMS = pltpu.CompilerParams(use_tc_tiling_on_sc=False,
                                  needs_layout_passes=False)


def _segsum_body(p_hbm, src_hbm, dst_hbm, zero_hbm, out_hbm,
                 src_v, dst_v, rows_v, acc_sh, sem):
    c = lax.axis_index("c")
    s = lax.axis_index("s")
    w = c * NS + s
    pltpu.sync_copy(zero_hbm, acc_sh.at[pl.ds(s * ZR, ZR)])
    pltpu.sync_copy(src_hbm.at[w], src_v)
    pltpu.sync_copy(dst_hbm.at[w], dst_v)
    plsc.subcore_barrier()

    def chunk(j, carry):
        pltpu.async_copy(p_hbm.at[src_v.at[j]], rows_v, sem).wait()
        pltpu.sync_copy(rows_v, acc_sh.at[dst_v.at[j]], add=True)
        return carry

    @pl.when(c == 0)
    def _():
        lax.fori_loop(0, CPW0, chunk, 0, unroll=False)

    @pl.when(c != 0)
    def _():
        lax.fori_loop(0, CPW1, chunk, 0, unroll=False)

    plsc.subcore_barrier()
    pltpu.sync_copy(acc_sh.at[pl.ds(s * ZR, ZR)],
                    out_hbm.at[c].at[pl.ds(s * ZR, ZR)])


@functools.cache
def _segsum_kernel():
    return pl.kernel(
        _segsum_body,
        out_type=jax.ShapeDtypeStruct((NC, N_ACC, WS), f32),
        mesh=_sc_mesh(),
        scratch_types=[
            pltpu.VMEM((CPW_MAX, CH), jnp.int32),
            pltpu.VMEM((CPW_MAX, CH), jnp.int32),
            pltpu.VMEM((CH, WS), f32),
            pltpu.VMEM_SHARED((N_ACC, WS), f32),
            pltpu.SemaphoreType.DMA,
        ],
        compiler_params=_SC_PARAMS,
    )


def _segsum(p, src, dst, zero):
    return _segsum_kernel()(p, src, dst, zero)


def _edgedot_body(zz_hbm, idx_hbm, out_hbm, idx_v, rows_v, sim_v, sem):
    c = lax.axis_index("c")
    s = lax.axis_index("s")
    w = c * NS + s
    pltpu.sync_copy(idx_hbm.at[w], idx_v)
    lane = lax.iota(jnp.int32, 16)

    def chunk(j, carry):
        pltpu.async_copy(zz_hbm.at[idx_v.at[j]], rows_v, sem).wait()
        for g in range(CH // 16):
            rows = lane + (g * 16)
            acc = jnp.zeros((16,), f32)
            for f in range(OH):
                col = jnp.full((16,), f, jnp.int32)
                acc = acc + (plsc.load_gather(rows_v, (rows, col))
                             * plsc.load_gather(rows_v, (rows + CH, col)))
            sim_v[pl.ds(g * 16, 16)] = 1.0 / (1.0 + jnp.exp(-acc))
        k = jnp.where(c == 0, j * NS + s, NS * CPW0 + j * NS + s)
        pltpu.sync_copy(sim_v, out_hbm.at[pl.ds(k * CH, CH)])
        return carry

    @pl.when(c == 0)
    def _():
        lax.fori_loop(0, CPW0, chunk, 0, unroll=False)

    @pl.when(c != 0)
    def _():
        lax.fori_loop(0, CPW1, chunk, 0, unroll=False)


@functools.cache
def _edgedot_kernel():
    return pl.kernel(
        _edgedot_body,
        out_type=jax.ShapeDtypeStruct((E_PAD,), f32),
        mesh=_sc_mesh(),
        scratch_types=[
            pltpu.VMEM((CPW_MAX, 2 * CH), jnp.int32),
            pltpu.VMEM((2 * CH, WS), f32),
            pltpu.VMEM((CH,), f32),
            pltpu.SemaphoreType.DMA,
        ],
        compiler_params=_SC_PARAMS,
    )


def _edgedot(zz, idx):
    return _edgedot_kernel()(zz, idx)


def _ones_col():
    col = lax.broadcasted_iota(jnp.int32, (1, WS), 1)
    return jnp.where(col == H, 1.0, 0.0).astype(f32)


def _prep0_body(z_ref, wl_ref, wr_ref, b_ref, p_ref, r_ref):
    zb = z_ref[...]
    p_ref[...] = jnp.dot(zb, wl_ref[...], preferred_element_type=f32) + _ones_col()
    r_ref[...] = jnp.dot(zb, wr_ref[...], preferred_element_type=f32) + b_ref[...]


def _combine(pa, pb, r):
    ssum = pa + pb
    col = lax.broadcasted_iota(jnp.int32, (1, WS), 1)
    cnt = jnp.sum(jnp.where(col == H, ssum, 0.0), axis=1, keepdims=True)
    mean = ssum / jnp.maximum(cnt, 1.0)
    return jnp.maximum(mean + r, 0.0)


def _comb_prep_body(pa_ref, pb_ref, r_ref, wl_ref, wr_ref, b_ref, p_ref, rn_ref):
    h = _combine(pa_ref[...], pb_ref[...], r_ref[...])
    p_ref[...] = jnp.dot(h, wl_ref[...], preferred_element_type=f32) + _ones_col()
    rn_ref[...] = jnp.dot(h, wr_ref[...], preferred_element_type=f32) + b_ref[...]


def _zz_body(pa_ref, pb_ref, r_ref, lw_ref, lb_ref, zz_ref):
    h = _combine(pa_ref[...], pb_ref[...], r_ref[...])
    zz_ref[...] = jnp.dot(h, lw_ref[...], preferred_element_type=f32) + lb_ref[...]


def _dec_body(z_ref, zz_ref, w0a_ref, w0b_ref, b0_ref, w1_ref, b1_ref,
              w2_ref, b2_ref, w3_ref, b3_ref, w4_ref, b4_ref, out_ref):
    x = jnp.maximum(jnp.dot(z_ref[...], w0a_ref[...], preferred_element_type=f32)
                    + jnp.dot(zz_ref[...], w0b_ref[...], preferred_element_type=f32)
                    + b0_ref[...], 0.0)
    for wr, br in ((w1_ref, b1_ref), (w2_ref, b2_ref), (w3_ref, b3_ref)):
        x = jnp.maximum(jnp.dot(x, wr[...], preferred_element_type=f32) + br[...], 0.0)
    lg = jnp.dot(x, w4_ref[...], preferred_element_type=f32) + b4_ref[...]
    col = lax.broadcasted_iota(jnp.int32, (1, W), 1)
    neg = jnp.where(col < XDIM, lg, -1e30)
    m = jnp.max(neg, axis=1, keepdims=True)
    ex = jnp.where(col < XDIM, jnp.exp(neg - m), 0.0)
    out_ref[...] = (neg - m) - jnp.log(jnp.sum(ex, axis=1, keepdims=True))


def _full(shape):
    return pl.BlockSpec(shape, lambda i: (0, 0))


def _rows(width):
    return pl.BlockSpec((BN, width), lambda i: (i, 0))


_GRID = (N // BN,)


def _call_prep0(z, wl, wr, b):
    return pl.pallas_call(
        _prep0_body, grid=_GRID,
        in_specs=[_rows(D), _full((D, WS)), _full((D, WS)), _full((1, WS))],
        out_specs=[_rows(WS), _rows(WS)],
        out_shape=[jax.ShapeDtypeStruct((N, WS), f32)] * 2,
    )(z, wl, wr, b)


def _call_comb_prep(pa, pb, r, wl, wr, b):
    return pl.pallas_call(
        _comb_prep_body, grid=_GRID,
        in_specs=[_rows(WS), _rows(WS), _rows(WS),
                  _full((WS, WS)), _full((WS, WS)), _full((1, WS))],
        out_specs=[_rows(WS), _rows(WS)],
        out_shape=[jax.ShapeDtypeStruct((N, WS), f32)] * 2,
    )(pa, pb, r, wl, wr, b)


def _call_zz(pa, pb, r, lw, lb):
    return pl.pallas_call(
        _zz_body, grid=_GRID,
        in_specs=[_rows(WS), _rows(WS), _rows(WS), _full((WS, WS)), _full((1, WS))],
        out_specs=_rows(WS),
        out_shape=jax.ShapeDtypeStruct((N, WS), f32),
    )(pa, pb, r, lw, lb)


def _call_dec(z, zz, w0a, w0b, b0, w1, b1, w2, b2, w3, b3, w4, b4):
    return pl.pallas_call(
        _dec_body, grid=_GRID,
        in_specs=[_rows(D), _rows(WS),
                  _full((D, W)), _full((WS, W)), _full((1, W)),
                  _full((W, W)), _full((1, W)),
                  _full((W, W)), _full((1, W)),
                  _full((W, W)), _full((1, W)),
                  _full((W, W)), _full((1, W))],
        out_specs=_rows(W),
        out_shape=jax.ShapeDtypeStruct((N, W), f32),
    )(z, zz, w0a, w0b, b0, w1, b1, w2, b2, w3, b3, w4, b4)


def _pad_w(w, rows, cols):
    return jnp.zeros((rows, cols), f32).at[:w.shape[0], :w.shape[1]].set(w)


def _pad_b(b, cols):
    return jnp.zeros((1, cols), f32).at[0, :b.shape[0]].set(b)


def _pad_flat(x, fill):
    pad = E_PAD - E
    return jnp.concatenate([x, jnp.full((pad,), 0, jnp.int32) + fill])


def _split_chunks(flat, ch):
    chunks = flat.reshape(CT, ch)
    n0 = NS * CPW0
    c0 = chunks[:n0].reshape(CPW0, NS, ch).transpose(1, 0, 2)
    c1 = chunks[n0:].reshape(CPW1, NS, ch).transpose(1, 0, 2)
    c1 = jnp.concatenate(
        [c1, jnp.zeros((NS, CPW_MAX - CPW1, ch), jnp.int32)], axis=1)
    return jnp.concatenate([c0, c1], axis=0)


def _prep_seg_edges(src, dst):
    pad = E_PAD - E
    fill = N + jnp.arange(pad, dtype=jnp.int32) % (N_ACC - N)
    return (_split_chunks(_pad_flat(src, 0), CH),
            _split_chunks(jnp.concatenate([dst, fill]), CH))


def _prep_dot_edges(src, dst):
    srcp = _pad_flat(src, 0).reshape(CT, CH)
    dstp = _pad_flat(dst, 0).reshape(CT, CH)
    comb = jnp.concatenate([srcp, dstp], axis=1).reshape(-1)
    return _split_chunks(comb, 2 * CH)


def kernel(z, edge_index, backbones, Wl0, Wr0, b0, Wl1, Wr1, b1, Wl2, Wr2, b2,
           linW, linB, dW0, db0, dW1, db1, dW2, db2, dW3, db3, dW4, db4):
    sb, db = _prep_seg_edges(backbones[0], backbones[1])
    ed = _prep_dot_edges(edge_index[0], edge_index[1])
    zero_rows = jnp.zeros((ZR, WS), f32)

    p, r = _call_prep0(z, _pad_w(Wl0, D, WS), _pad_w(Wr0, D, WS), _pad_b(b0, WS))
    part = _segsum(p, sb, db, zero_rows)
    p, r = _call_comb_prep(part[0], part[1], r,
                           _pad_w(Wl1, WS, WS), _pad_w(Wr1, WS, WS), _pad_b(b1, WS))
    part = _segsum(p, sb, db, zero_rows)
    p, r = _call_comb_prep(part[0], part[1], r,
                           _pad_w(Wl2, WS, WS), _pad_w(Wr2, WS, WS), _pad_b(b2, WS))
    part = _segsum(p, sb, db, zero_rows)
    zz = _call_zz(part[0], part[1], r, _pad_w(linW, WS, WS), _pad_b(linB, WS))

    sim = _edgedot(zz, ed)
    x_r = _call_dec(z, zz,
                    _pad_w(dW0[:D], D, W), _pad_w(dW0[D:], WS, W), _pad_b(db0, W),
                    _pad_w(dW1, W, W), _pad_b(db1, W),
                    _pad_w(dW2, W, W), _pad_b(db2, W),
                    _pad_w(dW3, W, W), _pad_b(db3, W),
                    _pad_w(dW4, W, W), _pad_b(db4, W))
    return (x_r[:, :XDIM], sim[:E])
